# baseline (speedup 1.0000x reference)
import functools

import jax
import jax.numpy as jnp
from jax import lax
from jax.experimental import pallas as pl
from jax.experimental.pallas import tpu as pltpu
from jax.experimental.pallas import tpu_sc as plsc

NC = 2
NS = 16
NW = NC * NS
LANES = 16

CH = 16
NBUF = 2
NACC = 2


def _sc_gather_reduce(idx_flat, pw, feats, n_pad, R, L, c_chunks):
  D = feats.shape[1]
  rpc = CH * R
  n_dc = D // LANES
  mesh = plsc.VectorSubcoreMesh(
      core_axis_name="c", subcore_axis_name="s",
      num_cores=NC, num_subcores=NS)

  @functools.partial(
      pl.kernel,
      out_type=jax.ShapeDtypeStruct((n_pad, D), jnp.float32),
      mesh=mesh,
      scratch_types=[
          pltpu.VMEM((L, D), jnp.float32),
          [pltpu.VMEM((rpc,), jnp.int32)] * NBUF,
          [pltpu.VMEM((rpc, D), jnp.float32)] * NBUF,
          [pltpu.VMEM((CH, D), jnp.float32)] * NACC,
          [pltpu.SemaphoreType.DMA] * NBUF,
          [pltpu.SemaphoreType.DMA] * NBUF,
          pltpu.SemaphoreType.DMA,
      ],
  )
  def sc_kernel(idx_hbm, pw_hbm, feats_hbm, acc_hbm,
                pw_v, idxs, rows, accs, isems, gsems, ssem):
    cid = lax.axis_index("c")
    sid = lax.axis_index("s")
    c0, c1 = c_chunks
    base_chunk = jnp.where(cid == 0, sid * c0, NS * c0 + sid * c1)
    my_chunks = jnp.where(cid == 0, c0, c1)
    n_outer = my_chunks // NBUF
    base_node = base_chunk * CH
    pltpu.sync_copy(pw_hbm, pw_v)

    def idx_load(c, b):
      return pltpu.make_async_copy(
          idx_hbm.at[pl.ds((base_chunk + c) * CH * R, rpc)], idxs[b], isems[b])

    def idx_start(c, b):
      idx_load(c, b).start()

    def idx_wait(c, b):
      idx_load(c, b).wait()

    def gather(b):
      return pltpu.make_async_copy(
          feats_hbm.at[idxs[b]], rows[b], gsems[b])

    def store(c, a):
      return pltpu.make_async_copy(
          accs[a], acc_hbm.at[pl.ds(base_node + c * CH, CH), :], ssem)

    for b in range(NBUF):
      idx_start(b, b)
    idx_wait(0, 0)
    gather(0).start()

    def outer(i, carry):
      for b in range(NBUF):
        j = i * NBUF + b
        b2 = (b + 1) % NBUF
        a = b % NACC
        gather(b).wait()
        @pl.when(i < n_outer - 1)
        def _refill():
          idx_start(j + NBUF, b)
        if b == 0:
          idx_wait(j + 1, b2)
          gather(b2).start()
        else:
          @pl.when(i < n_outer - 1)
          def _fire():
            idx_wait(j + 1, b2)
            gather(b2).start()
        @pl.when(i > 0)
        def _drain():
          store(j - NACC, a).wait()

        for dc in range(n_dc):
          dsl = pl.ds(dc * LANES, LANES)
          w_regs = [pw_v[l, dsl] for l in range(L)]

          def t_body(t, c2, _b=b, _a=a, _dsl=dsl, _w=w_regs):
            r0 = t * R
            x = rows[_b][r0, _dsl] * _w[0]
            for r in range(1, R):
              x = x + rows[_b][r0 + r, _dsl] * _w[r % L]
            accs[_a][t, _dsl] = x
            return c2

          lax.fori_loop(0, CH, t_body, 0)

        store(j, a).start()
      return carry

    lax.fori_loop(0, n_outer, outer, 0)
    for b in range(NACC):
      store(my_chunks - NACC + b, b).wait()

  return sc_kernel(idx_flat, pw, feats)


def _tc_matmul_relu(acc, wt, n_out):
  D = wt.shape[0]
  blk = 2000
  grid = n_out // blk

  def mm_body(x_ref, wt_ref, o_ref):
    o_ref[...] = jnp.maximum(
        jnp.dot(x_ref[...], wt_ref[...], preferred_element_type=jnp.float32),
        0.0)

  return pl.pallas_call(
      mm_body,
      grid=(grid,),
      in_specs=[
          pl.BlockSpec((blk, D), lambda i: (i, 0)),
          pl.BlockSpec((D, D), lambda i: (0, 0)),
      ],
      out_specs=pl.BlockSpec((blk, D), lambda i: (i, 0)),
      out_shape=jax.ShapeDtypeStruct((n_out, D), jnp.float32),
  )(acc, wt)


def kernel(feats, paths, init_feats, flag, path_weight1, path_weight2, W):
  del init_feats
  N, D = feats.shape
  P, _, L = paths.shape
  R = P * L

  if path_weight1.shape[1] == L:
    pw = jnp.where(flag == 1, path_weight1,
                   path_weight1 + 0.0 * path_weight2.sum())
  else:
    pw = jnp.where(flag == 1, path_weight2,
                   path_weight2 + 0.0 * path_weight1.sum())
  pw = (pw[0] * (1.0 / P)).astype(jnp.float32)

  total_chunks = -(-(-(-N // (NS * CH))) // NBUF) * NBUF
  c0 = (total_chunks * 29 // 50) // NBUF * NBUF
  c1 = total_chunks - c0
  n_pad = NS * total_chunks * CH

  idx_flat = jnp.transpose(paths, (1, 0, 2)).reshape(-1)
  idx_flat = jnp.pad(idx_flat, (0, (n_pad - N) * R))

  acc = _sc_gather_reduce(idx_flat, pw, feats, n_pad, R, L, (c0, c1))
  return _tc_matmul_relu(acc, W.T, N)


# speedup vs baseline: 1.3179x; 1.3179x over previous
import functools

import jax
import jax.numpy as jnp
from jax import lax
from jax.experimental import pallas as pl
from jax.experimental.pallas import tpu as pltpu
from jax.experimental.pallas import tpu_sc as plsc

NC = 2
NS = 16
NW = NC * NS
LANES = 16

CH = 16
NBUF = 2
NACC = 2


def _sc_gather_reduce(idx_flat, pw, feats, n_pad, R, L, c_chunks):
  D = feats.shape[1]
  rpc = CH * R
  n_dc = D // LANES
  mesh = plsc.VectorSubcoreMesh(
      core_axis_name="c", subcore_axis_name="s",
      num_cores=NC, num_subcores=NS)

  @functools.partial(
      pl.kernel,
      out_type=jax.ShapeDtypeStruct((n_pad, D), jnp.float32),
      mesh=mesh,
      scratch_types=[
          pltpu.VMEM((L, D), jnp.float32),
          [pltpu.VMEM((rpc,), jnp.int32)] * NBUF,
          [pltpu.VMEM((rpc, D), jnp.float32)] * NBUF,
          [pltpu.VMEM((CH, D), jnp.float32)] * NACC,
          [pltpu.SemaphoreType.DMA] * NBUF,
          [pltpu.SemaphoreType.DMA] * NBUF,
          pltpu.SemaphoreType.DMA,
      ],
  )
  def sc_kernel(idx_hbm, pw_hbm, feats_hbm, acc_hbm,
                pw_v, idxs, rows, accs, isems, gsems, ssem):
    cid = lax.axis_index("c")
    sid = lax.axis_index("s")
    c0, c1 = c_chunks
    base_chunk = jnp.where(cid == 0, sid * c0, NS * c0 + sid * c1)
    my_chunks = jnp.where(cid == 0, c0, c1)
    n_outer = my_chunks // NBUF
    base_node = base_chunk * CH
    pltpu.sync_copy(pw_hbm, pw_v)

    def idx_load(c, b):
      return pltpu.make_async_copy(
          idx_hbm.at[pl.ds((base_chunk + c) * CH * R, rpc)], idxs[b], isems[b])

    def idx_start(c, b):
      idx_load(c, b).start()

    def idx_wait(c, b):
      idx_load(c, b).wait()

    def gather(b):
      return pltpu.make_async_copy(
          feats_hbm.at[idxs[b]], rows[b], gsems[b])

    def store(c, a):
      return pltpu.make_async_copy(
          accs[a], acc_hbm.at[pl.ds(base_node + c * CH, CH), :], ssem)

    for b in range(NBUF):
      idx_start(b, b)
    idx_wait(0, 0)
    gather(0).start()

    def outer(i, carry):
      for b in range(NBUF):
        j = i * NBUF + b
        b2 = (b + 1) % NBUF
        a = b % NACC
        gather(b).wait()
        @pl.when(i < n_outer - 1)
        def _refill():
          idx_start(j + NBUF, b)
        if b == 0:
          idx_wait(j + 1, b2)
          gather(b2).start()
        else:
          @pl.when(i < n_outer - 1)
          def _fire():
            idx_wait(j + 1, b2)
            gather(b2).start()
        @pl.when(i > 0)
        def _drain():
          store(j - NACC, a).wait()

        for dc in range(n_dc):
          dsl = pl.ds(dc * LANES, LANES)
          w_regs = [pw_v[l, dsl] for l in range(L)]

          def t_body(t, c2, _b=b, _a=a, _dsl=dsl, _w=w_regs):
            r0 = t * R
            x = rows[_b][r0, _dsl] * _w[0]
            for r in range(1, R):
              x = x + rows[_b][r0 + r, _dsl] * _w[r % L]
            accs[_a][t, _dsl] = x
            return c2

          lax.fori_loop(0, CH, t_body, 0)

        store(j, a).start()
      return carry

    lax.fori_loop(0, n_outer, outer, 0)
    for b in range(NACC):
      store(my_chunks - NACC + b, b).wait()

  return sc_kernel(idx_flat, pw, feats)


def _tc_matmul_relu(acc, wt, n_out):
  D = wt.shape[0]
  blk = 2000
  grid = n_out // blk

  def mm_body(x_ref, wt_ref, o_ref):
    o_ref[...] = jnp.maximum(
        jnp.dot(x_ref[...], wt_ref[...], preferred_element_type=jnp.float32),
        0.0)

  return pl.pallas_call(
      mm_body,
      grid=(grid,),
      in_specs=[
          pl.BlockSpec((blk, D), lambda i: (i, 0)),
          pl.BlockSpec((D, D), lambda i: (0, 0)),
      ],
      out_specs=pl.BlockSpec((blk, D), lambda i: (i, 0)),
      out_shape=jax.ShapeDtypeStruct((n_out, D), jnp.float32),
  )(acc, wt)


def kernel(feats, paths, init_feats, flag, path_weight1, path_weight2, W):
  del init_feats
  N, D = feats.shape
  P, _, L = paths.shape
  R = P * L

  if path_weight1.shape[1] == L:
    pw = jnp.where(flag == 1, path_weight1,
                   path_weight1 + 0.0 * path_weight2.sum())
  else:
    pw = jnp.where(flag == 1, path_weight2,
                   path_weight2 + 0.0 * path_weight1.sum())
  pw = (pw[0] * (1.0 / P)).astype(jnp.float32)

  total_chunks = -(-(-(-N // (NS * CH))) // NBUF) * NBUF
  c0 = (total_chunks * 29 // 50) // NBUF * NBUF
  c1 = total_chunks - c0
  n_pad = NS * total_chunks * CH

  idx = jnp.transpose(paths, (1, 0, 2)).reshape(N, R)
  idx = jnp.pad(idx, ((0, n_pad - N), (0, 0)))
  idx_flat = idx.reshape(-1)

  acc = _sc_gather_reduce(idx_flat, pw, feats, n_pad, R, L, (c0, c1))
  return _tc_matmul_relu(acc, W.T, N)


# speedup vs baseline: 1.3309x; 1.0099x over previous
import functools

import jax
import jax.numpy as jnp
from jax import lax
from jax.experimental import pallas as pl
from jax.experimental.pallas import tpu as pltpu
from jax.experimental.pallas import tpu_sc as plsc

NC = 2
NS = 16
NW = NC * NS
LANES = 16

CH = 16
NBUF = 2
NACC = 2


def _sc_gather_reduce(idx_flat, pw, feats, n_pad, R, L, c_chunks):
  D = feats.shape[1]
  rpc = CH * R
  n_dc = D // LANES
  mesh = plsc.VectorSubcoreMesh(
      core_axis_name="c", subcore_axis_name="s",
      num_cores=NC, num_subcores=NS)

  @functools.partial(
      pl.kernel,
      out_type=jax.ShapeDtypeStruct((n_pad, D), jnp.float32),
      mesh=mesh,
      scratch_types=[
          pltpu.VMEM((L, D), jnp.float32),
          [pltpu.VMEM((rpc,), jnp.int32)] * NBUF,
          [pltpu.VMEM((rpc, D), jnp.float32)] * NBUF,
          [pltpu.VMEM((CH, D), jnp.float32)] * NACC,
          [pltpu.SemaphoreType.DMA] * NBUF,
          [pltpu.SemaphoreType.DMA] * NBUF,
          pltpu.SemaphoreType.DMA,
      ],
  )
  def sc_kernel(idx_hbm, pw_hbm, feats_hbm, acc_hbm,
                pw_v, idxs, rows, accs, isems, gsems, ssem):
    cid = lax.axis_index("c")
    sid = lax.axis_index("s")
    c0, c1 = c_chunks
    base_chunk = jnp.where(cid == 0, sid * c0, NS * c0 + sid * c1)
    my_chunks = jnp.where(cid == 0, c0, c1)
    n_outer = my_chunks // NBUF
    base_node = base_chunk * CH
    pltpu.sync_copy(pw_hbm, pw_v)

    def idx_load(c, b):
      return pltpu.make_async_copy(
          idx_hbm.at[pl.ds((base_chunk + c) * CH * R, rpc)], idxs[b], isems[b])

    def idx_start(c, b):
      idx_load(c, b).start()

    def idx_wait(c, b):
      idx_load(c, b).wait()

    def gather(b):
      return pltpu.make_async_copy(
          feats_hbm.at[idxs[b]], rows[b], gsems[b])

    def store(c, a):
      return pltpu.make_async_copy(
          accs[a], acc_hbm.at[pl.ds(base_node + c * CH, CH), :], ssem)

    for b in range(NBUF):
      idx_start(b, b)
    idx_wait(0, 0)
    gather(0).start()

    def outer(i, carry):
      for b in range(NBUF):
        j = i * NBUF + b
        b2 = (b + 1) % NBUF
        a = b % NACC
        gather(b).wait()
        @pl.when(i < n_outer - 1)
        def _refill():
          idx_start(j + NBUF, b)
        if b == 0:
          idx_wait(j + 1, b2)
          gather(b2).start()
        else:
          @pl.when(i < n_outer - 1)
          def _fire():
            idx_wait(j + 1, b2)
            gather(b2).start()
        @pl.when(i > 0)
        def _drain():
          store(j - NACC, a).wait()

        for dc in range(n_dc):
          dsl = pl.ds(dc * LANES, LANES)
          w_regs = [pw_v[l, dsl] for l in range(L)]

          def t_body(t, c2, _b=b, _a=a, _dsl=dsl, _w=w_regs):
            r0 = t * R
            x = rows[_b][r0, _dsl] * _w[0]
            for r in range(1, R):
              x = x + rows[_b][r0 + r, _dsl] * _w[r % L]
            accs[_a][t, _dsl] = x
            return c2

          lax.fori_loop(0, CH, t_body, 0)

        store(j, a).start()
      return carry

    lax.fori_loop(0, n_outer, outer, 0)
    for b in range(NACC):
      store(my_chunks - NACC + b, b).wait()

  return sc_kernel(idx_flat, pw, feats)


def _tc_matmul_relu(acc, wt, n_out):
  D = wt.shape[0]
  blk = 2000
  grid = n_out // blk

  def mm_body(x_ref, wt_ref, o_ref):
    o_ref[...] = jnp.maximum(
        jnp.dot(x_ref[...], wt_ref[...], preferred_element_type=jnp.float32),
        0.0)

  return pl.pallas_call(
      mm_body,
      grid=(grid,),
      in_specs=[
          pl.BlockSpec((blk, D), lambda i: (i, 0)),
          pl.BlockSpec((D, D), lambda i: (0, 0)),
      ],
      out_specs=pl.BlockSpec((blk, D), lambda i: (i, 0)),
      out_shape=jax.ShapeDtypeStruct((n_out, D), jnp.float32),
  )(acc, wt)


def kernel(feats, paths, init_feats, flag, path_weight1, path_weight2, W):
  del init_feats
  N, D = feats.shape
  P, _, L = paths.shape
  R = P * L

  if path_weight1.shape[1] == L:
    pw = jnp.where(flag == 1, path_weight1,
                   path_weight1 + 0.0 * path_weight2.sum())
  else:
    pw = jnp.where(flag == 1, path_weight2,
                   path_weight2 + 0.0 * path_weight1.sum())
  pw = (pw[0] * (1.0 / P)).astype(jnp.float32)

  total_chunks = -(-(-(-N // (NS * CH))) // NBUF) * NBUF
  c0 = (total_chunks * 119 // 200) // NBUF * NBUF
  c1 = total_chunks - c0
  n_pad = NS * total_chunks * CH

  idx = jnp.transpose(paths, (1, 0, 2)).reshape(N, R)
  idx = jnp.pad(idx, ((0, n_pad - N), (0, 0)))
  idx_flat = idx.reshape(-1)

  acc = _sc_gather_reduce(idx_flat, pw, feats, n_pad, R, L, (c0, c1))
  return _tc_matmul_relu(acc, W.T, N)
